# baseline (speedup 1.0000x reference)
import functools

import jax
import jax.numpy as jnp
from jax import lax
from jax.experimental import pallas as pl
from jax.experimental.pallas import tpu as pltpu
from jax.experimental.pallas import tpu_sc as plsc

B, T, D = 8, 1024, 256
G, K = 8, 8192
GD = D // G
N = B * T
TN = 512
NB = N // TN
TK = 1024
NKC = K // TK


def _vq_argmin_kernel(zt_ref, cb_ref, zn_ref, cbn_ref, idx_ref, fidx_ref,
                      loss_ref):
    g = pl.program_id(0)
    nb = pl.program_id(1)
    z = zt_ref[0]
    zn = zn_ref[0, 0, :][:, None]

    def body(c, carry):
        rmin, rarg = carry
        cbc = cb_ref[0, pl.ds(c * TK, TK), :]
        prod = lax.dot_general(z.astype(jnp.bfloat16), cbc.astype(jnp.bfloat16),
                               (((1,), (1,)), ((), ())),
                               preferred_element_type=jnp.float32)
        cbn = cbn_ref[0, 0, pl.ds(c * TK, TK)][None, :]
        dist = (zn - 2.0 * prod) + cbn
        m = jnp.min(dist, axis=1)
        iota = lax.broadcasted_iota(jnp.int32, (TN, TK), 1)
        cand = jnp.where(dist == m[:, None], iota, K)
        a = jnp.min(cand, axis=1) + c * TK
        rmin = jnp.where(c == NKC // 2,
                         rmin.astype(jnp.bfloat16).astype(jnp.float32), rmin)
        better = m < rmin
        return (jnp.where(better, m, rmin), jnp.where(better, a, rarg))

    rmin0 = jnp.full((TN,), jnp.inf, jnp.float32)
    rarg0 = jnp.zeros((TN,), jnp.int32)
    rmin, rarg = lax.fori_loop(0, NKC, body, (rmin0, rarg0))

    idx_ref[0, 0, :] = rarg
    fidx_ref[0, 0, :] = rarg + g * K

    first = jnp.logical_and(g == 0, nb == 0)
    last = jnp.logical_and(g == G - 1, nb == NB - 1)
    prev = jnp.where(first, jnp.zeros((1, 1), jnp.float32), loss_ref[...])
    total = prev + jnp.sum(rmin)
    loss_ref[...] = jnp.where(last, total * (2.0 / (N * GD)), total)


def _run_argmin(zt, codebooks, zn, cbn):
    return pl.pallas_call(
        _vq_argmin_kernel,
        grid=(G, NB),
        in_specs=[
            pl.BlockSpec((1, TN, GD), lambda g, nb: (g, nb, 0)),
            pl.BlockSpec((1, K, GD), lambda g, nb: (g, 0, 0)),
            pl.BlockSpec((1, 1, TN), lambda g, nb: (g * NB + nb, 0, 0)),
            pl.BlockSpec((1, 1, K), lambda g, nb: (g, 0, 0)),
        ],
        out_specs=[
            pl.BlockSpec((1, 1, TN), lambda g, nb: (g * NB + nb, 0, 0)),
            pl.BlockSpec((1, 1, TN), lambda g, nb: (g * NB + nb, 0, 0)),
            pl.BlockSpec((1, 1), lambda g, nb: (0, 0)),
        ],
        out_shape=[
            jax.ShapeDtypeStruct((G * NB, 1, TN), jnp.int32),
            jax.ShapeDtypeStruct((G * NB, 1, TN), jnp.int32),
            jax.ShapeDtypeStruct((1, 1), jnp.float32),
        ],
    )(zt, codebooks, zn, cbn)


NG = N * G
_NC = 2
_NW = 32
RPW = NG // _NW
CH = 128
NCH = RPW // CH


@functools.cache
def _make_sc_gather():
    @functools.partial(
        pl.kernel,
        mesh=plsc.VectorSubcoreMesh(core_axis_name="c", subcore_axis_name="s"),
        out_type=jax.ShapeDtypeStruct((NG, GD), jnp.float32),
        scratch_types=[
            pltpu.VMEM((NCH, CH), jnp.int32),
            pltpu.VMEM((RPW, GD), jnp.float32),
            pltpu.SemaphoreType.DMA,
        ],
        compiler_params=pltpu.CompilerParams(use_tc_tiling_on_sc=False),
    )
    def _sc_gather(table_hbm, fidx_hbm, out_hbm, idx_v, rows_v, sem):
        wid = lax.axis_index("s") * _NC + lax.axis_index("c")
        pltpu.sync_copy(fidx_hbm.at[pl.ds(wid * NCH, NCH)], idx_v)
        copies = []
        for j in range(NCH):
            copies.append(
                pltpu.async_copy(table_hbm.at[idx_v.at[j]],
                                 rows_v.at[pl.ds(j * CH, CH)], sem))
        for c in copies:
            c.wait()
        pltpu.sync_copy(rows_v, out_hbm.at[pl.ds(wid * RPW, RPW)])

    return _sc_gather


def kernel(z_e, codebooks):
    zt = z_e.reshape(N, G, GD).transpose(1, 0, 2)
    zn = jnp.sum(zt * zt, axis=2).reshape(G * NB, 1, TN)
    cbn = jnp.sum(codebooks * codebooks, axis=2).reshape(G, 1, K)
    idx_g, fidx_g, loss = _run_argmin(zt, codebooks, zn, cbn)
    idx = idx_g.reshape(G, N).T
    fidx = fidx_g.reshape(G, N).T.reshape(_NW * NCH, CH)
    table = codebooks.reshape(G * K, GD)
    zq = _make_sc_gather()(table, fidx)
    return (zq.reshape(B, T, D), loss[0, 0], idx.reshape(B, T, G))


# speedup vs baseline: 1.1690x; 1.1690x over previous
import functools

import jax
import jax.numpy as jnp
from jax import lax
from jax.experimental import pallas as pl
from jax.experimental.pallas import tpu as pltpu
from jax.experimental.pallas import tpu_sc as plsc

B, T, D = 8, 1024, 256
G, K = 8, 8192
GD = D // G
N = B * T
TN = 512
NB = N // TN
TK = 1024
NKC = K // TK


def _vq_argmin_kernel(zt_ref, cb_ref, zn_ref, cbn_ref, idx_ref, fidx_ref,
                      loss_ref):
    g = pl.program_id(0)
    nb = pl.program_id(1)
    z = zt_ref[0]
    zb = z.astype(jnp.bfloat16)
    zn = zn_ref[0, 0, :][None, :]

    def body(c, carry):
        rmin, rarg = carry
        cbc = cb_ref[0, pl.ds(c * TK, TK), :]
        prod = lax.dot_general(cbc.astype(jnp.bfloat16), zb,
                               (((1,), (1,)), ((), ())),
                               preferred_element_type=jnp.float32)
        cbn = cbn_ref[0, 0, pl.ds(c * TK, TK)][:, None]
        dist = (zn - 2.0 * prod) + cbn
        m = jnp.min(dist, axis=0)
        iota = lax.broadcasted_iota(jnp.int32, (TK, TN), 0)
        cand = jnp.where(dist == m[None, :], iota, K)
        a = jnp.min(cand, axis=0) + c * TK
        rmin = jnp.where(c == NKC // 2,
                         rmin.astype(jnp.bfloat16).astype(jnp.float32), rmin)
        better = m < rmin
        return (jnp.where(better, m, rmin), jnp.where(better, a, rarg))

    rmin0 = jnp.full((TN,), jnp.inf, jnp.float32)
    rarg0 = jnp.zeros((TN,), jnp.int32)
    rmin, rarg = lax.fori_loop(0, NKC, body, (rmin0, rarg0))

    idx_ref[0, 0, :] = rarg
    fidx_ref[0, 0, :] = rarg + g * K

    first = jnp.logical_and(g == 0, nb == 0)
    last = jnp.logical_and(g == G - 1, nb == NB - 1)
    prev = jnp.where(first, jnp.zeros((1, 1), jnp.float32), loss_ref[...])
    total = prev + jnp.sum(rmin)
    loss_ref[...] = jnp.where(last, total * (2.0 / (N * GD)), total)


def _run_argmin(zt, codebooks, zn, cbn):
    return pl.pallas_call(
        _vq_argmin_kernel,
        grid=(G, NB),
        in_specs=[
            pl.BlockSpec((1, TN, GD), lambda g, nb: (g, nb, 0)),
            pl.BlockSpec((1, K, GD), lambda g, nb: (g, 0, 0)),
            pl.BlockSpec((1, 1, TN), lambda g, nb: (g * NB + nb, 0, 0)),
            pl.BlockSpec((1, 1, K), lambda g, nb: (g, 0, 0)),
        ],
        out_specs=[
            pl.BlockSpec((1, 1, TN), lambda g, nb: (g * NB + nb, 0, 0)),
            pl.BlockSpec((1, 1, TN), lambda g, nb: (g * NB + nb, 0, 0)),
            pl.BlockSpec((1, 1), lambda g, nb: (0, 0)),
        ],
        out_shape=[
            jax.ShapeDtypeStruct((G * NB, 1, TN), jnp.int32),
            jax.ShapeDtypeStruct((G * NB, 1, TN), jnp.int32),
            jax.ShapeDtypeStruct((1, 1), jnp.float32),
        ],
    )(zt, codebooks, zn, cbn)


NG = N * G
_NC = 2
_NW = 32
RPW = NG // _NW
CH = 128
NCH = RPW // CH


@functools.cache
def _make_sc_gather():
    @functools.partial(
        pl.kernel,
        mesh=plsc.VectorSubcoreMesh(core_axis_name="c", subcore_axis_name="s"),
        out_type=jax.ShapeDtypeStruct((NG, GD), jnp.float32),
        scratch_types=[
            pltpu.VMEM((NCH, CH), jnp.int32),
            pltpu.VMEM((RPW, GD), jnp.float32),
            pltpu.SemaphoreType.DMA,
        ],
        compiler_params=pltpu.CompilerParams(use_tc_tiling_on_sc=False),
    )
    def _sc_gather(table_hbm, fidx_hbm, out_hbm, idx_v, rows_v, sem):
        wid = lax.axis_index("s") * _NC + lax.axis_index("c")
        pltpu.sync_copy(fidx_hbm.at[pl.ds(wid * NCH, NCH)], idx_v)
        copies = []
        for j in range(NCH):
            copies.append(
                pltpu.async_copy(table_hbm.at[idx_v.at[j]],
                                 rows_v.at[pl.ds(j * CH, CH)], sem))
        for c in copies:
            c.wait()
        pltpu.sync_copy(rows_v, out_hbm.at[pl.ds(wid * RPW, RPW)])

    return _sc_gather


def kernel(z_e, codebooks):
    zt = z_e.reshape(N, G, GD).transpose(1, 0, 2)
    zn = jnp.sum(zt * zt, axis=2).reshape(G * NB, 1, TN)
    cbn = jnp.sum(codebooks * codebooks, axis=2).reshape(G, 1, K)
    idx_g, fidx_g, loss = _run_argmin(zt, codebooks, zn, cbn)
    idx = idx_g.reshape(G, N).T
    fidx = fidx_g.reshape(G, N).T.reshape(_NW * NCH, CH)
    table = codebooks.reshape(G * K, GD)
    zq = _make_sc_gather()(table, fidx)
    return (zq.reshape(B, T, D), loss[0, 0], idx.reshape(B, T, G))


# speedup vs baseline: 1.3984x; 1.1962x over previous
import functools

import jax
import jax.numpy as jnp
from jax import lax
from jax.experimental import pallas as pl
from jax.experimental.pallas import tpu as pltpu
from jax.experimental.pallas import tpu_sc as plsc

B, T, D = 8, 1024, 256
G, K = 8, 8192
GD = D // G
N = B * T
TN = 512
NB = N // TN
TK = 1024
NKC = K // TK


def _vq_argmin_kernel(zt_ref, cb_ref, zn_ref, cbn_ref, idx_ref, fidx_ref,
                      loss_ref):
    g = pl.program_id(0)
    nb = pl.program_id(1)
    z = zt_ref[0]
    zb = z.astype(jnp.bfloat16)
    zn = zn_ref[0, 0, :][None, :]

    def body(c, carry):
        rmin, rarg = carry
        cbc = cb_ref[0, pl.ds(c * TK, TK), :]
        prod = lax.dot_general(cbc.astype(jnp.bfloat16), zb,
                               (((1,), (1,)), ((), ())),
                               preferred_element_type=jnp.float32)
        cbn = cbn_ref[0, 0, pl.ds(c * TK, TK)][:, None]
        dist = (zn - 2.0 * prod) + cbn
        m = jnp.min(dist, axis=0)
        a = jnp.argmin(dist, axis=0).astype(jnp.int32) + c * TK
        rmin = jnp.where(c == NKC // 2,
                         rmin.astype(jnp.bfloat16).astype(jnp.float32), rmin)
        better = m < rmin
        return (jnp.where(better, m, rmin), jnp.where(better, a, rarg))

    rmin0 = jnp.full((TN,), jnp.inf, jnp.float32)
    rarg0 = jnp.zeros((TN,), jnp.int32)
    rmin, rarg = lax.fori_loop(0, NKC, body, (rmin0, rarg0))

    idx_ref[0, 0, :] = rarg
    fidx_ref[0, 0, :] = rarg + g * K

    first = jnp.logical_and(g == 0, nb == 0)
    last = jnp.logical_and(g == G - 1, nb == NB - 1)
    prev = jnp.where(first, jnp.zeros((1, 1), jnp.float32), loss_ref[...])
    total = prev + jnp.sum(rmin)
    loss_ref[...] = jnp.where(last, total * (2.0 / (N * GD)), total)


def _run_argmin(zt, codebooks, zn, cbn):
    return pl.pallas_call(
        _vq_argmin_kernel,
        grid=(G, NB),
        in_specs=[
            pl.BlockSpec((1, TN, GD), lambda g, nb: (g, nb, 0)),
            pl.BlockSpec((1, K, GD), lambda g, nb: (g, 0, 0)),
            pl.BlockSpec((1, 1, TN), lambda g, nb: (g * NB + nb, 0, 0)),
            pl.BlockSpec((1, 1, K), lambda g, nb: (g, 0, 0)),
        ],
        out_specs=[
            pl.BlockSpec((1, 1, TN), lambda g, nb: (g * NB + nb, 0, 0)),
            pl.BlockSpec((1, 1, TN), lambda g, nb: (g * NB + nb, 0, 0)),
            pl.BlockSpec((1, 1), lambda g, nb: (0, 0)),
        ],
        out_shape=[
            jax.ShapeDtypeStruct((G * NB, 1, TN), jnp.int32),
            jax.ShapeDtypeStruct((G * NB, 1, TN), jnp.int32),
            jax.ShapeDtypeStruct((1, 1), jnp.float32),
        ],
    )(zt, codebooks, zn, cbn)


NG = N * G
_NC = 2
_NW = 32
RPW = NG // _NW
CH = 128
NCH = RPW // CH


@functools.cache
def _make_sc_gather():
    @functools.partial(
        pl.kernel,
        mesh=plsc.VectorSubcoreMesh(core_axis_name="c", subcore_axis_name="s"),
        out_type=jax.ShapeDtypeStruct((NG, GD), jnp.float32),
        scratch_types=[
            pltpu.VMEM((NCH, CH), jnp.int32),
            pltpu.VMEM((RPW, GD), jnp.float32),
            pltpu.SemaphoreType.DMA,
        ],
        compiler_params=pltpu.CompilerParams(use_tc_tiling_on_sc=False),
    )
    def _sc_gather(table_hbm, fidx_hbm, out_hbm, idx_v, rows_v, sem):
        wid = lax.axis_index("s") * _NC + lax.axis_index("c")
        pltpu.sync_copy(fidx_hbm.at[pl.ds(wid * NCH, NCH)], idx_v)
        copies = []
        for j in range(NCH):
            copies.append(
                pltpu.async_copy(table_hbm.at[idx_v.at[j]],
                                 rows_v.at[pl.ds(j * CH, CH)], sem))
        for c in copies:
            c.wait()
        pltpu.sync_copy(rows_v, out_hbm.at[pl.ds(wid * RPW, RPW)])

    return _sc_gather


def kernel(z_e, codebooks):
    zt = z_e.reshape(N, G, GD).transpose(1, 0, 2)
    zn = jnp.sum(zt * zt, axis=2).reshape(G * NB, 1, TN)
    cbn = jnp.sum(codebooks * codebooks, axis=2).reshape(G, 1, K)
    idx_g, fidx_g, loss = _run_argmin(zt, codebooks, zn, cbn)
    idx = idx_g.reshape(G, N).T
    fidx = fidx_g.reshape(G, N).T.reshape(_NW * NCH, CH)
    table = codebooks.reshape(G * K, GD)
    zq = _make_sc_gather()(table, fidx)
    return (zq.reshape(B, T, D), loss[0, 0], idx.reshape(B, T, G))


# speedup vs baseline: 1.7807x; 1.2734x over previous
import functools

import jax
import jax.numpy as jnp
from jax import lax
from jax.experimental import pallas as pl
from jax.experimental.pallas import tpu as pltpu
from jax.experimental.pallas import tpu_sc as plsc

B, T, D = 8, 1024, 256
G, K = 8, 8192
GD = D // G
N = B * T
TN = 512
NB = N // TN
TK = 1024
NKC = K // TK


def _vq_argmin_kernel(zt_ref, cb_ref, zn_ref, cbn_ref, idx_ref, fidx_ref,
                      loss_ref):
    g = pl.program_id(0)
    nb = pl.program_id(1)
    z = zt_ref[0]
    zb = z.astype(jnp.bfloat16)
    zn = zn_ref[0, 0, :][None, :]

    def body(c, carry):
        rmin, rarg = carry
        cbc = cb_ref[0, pl.ds(c * TK, TK), :]
        prod = lax.dot_general(cbc.astype(jnp.bfloat16), zb,
                               (((1,), (1,)), ((), ())),
                               preferred_element_type=jnp.float32)
        cbn = cbn_ref[0, 0, pl.ds(c * TK, TK)][:, None]
        dist = (zn - 2.0 * prod) + cbn
        m = jnp.min(dist, axis=0)
        a = jnp.argmin(dist, axis=0).astype(jnp.int32) + c * TK
        rmin = jnp.where(c == NKC // 2,
                         rmin.astype(jnp.bfloat16).astype(jnp.float32), rmin)
        better = m < rmin
        return (jnp.where(better, m, rmin), jnp.where(better, a, rarg))

    rmin0 = jnp.full((TN,), jnp.inf, jnp.float32)
    rarg0 = jnp.zeros((TN,), jnp.int32)
    carry = (rmin0, rarg0)
    for c in range(NKC):
        carry = body(c, carry)
    rmin, rarg = carry

    idx_ref[0, 0, :] = rarg
    fidx_ref[0, 0, :] = rarg + g * K

    first = jnp.logical_and(g == 0, nb == 0)
    last = jnp.logical_and(g == G - 1, nb == NB - 1)
    prev = jnp.where(first, jnp.zeros((1, 1), jnp.float32), loss_ref[...])
    total = prev + jnp.sum(rmin)
    loss_ref[...] = jnp.where(last, total * (2.0 / (N * GD)), total)


def _run_argmin(zt, codebooks, zn, cbn):
    return pl.pallas_call(
        _vq_argmin_kernel,
        grid=(G, NB),
        in_specs=[
            pl.BlockSpec((1, TN, GD), lambda g, nb: (g, nb, 0)),
            pl.BlockSpec((1, K, GD), lambda g, nb: (g, 0, 0)),
            pl.BlockSpec((1, 1, TN), lambda g, nb: (g * NB + nb, 0, 0)),
            pl.BlockSpec((1, 1, K), lambda g, nb: (g, 0, 0)),
        ],
        out_specs=[
            pl.BlockSpec((1, 1, TN), lambda g, nb: (g * NB + nb, 0, 0)),
            pl.BlockSpec((1, 1, TN), lambda g, nb: (g * NB + nb, 0, 0)),
            pl.BlockSpec((1, 1), lambda g, nb: (0, 0)),
        ],
        out_shape=[
            jax.ShapeDtypeStruct((G * NB, 1, TN), jnp.int32),
            jax.ShapeDtypeStruct((G * NB, 1, TN), jnp.int32),
            jax.ShapeDtypeStruct((1, 1), jnp.float32),
        ],
    )(zt, codebooks, zn, cbn)


NG = N * G
_NC = 2
_NW = 32
RPW = NG // _NW
CH = 128
NCH = RPW // CH


@functools.cache
def _make_sc_gather():
    @functools.partial(
        pl.kernel,
        mesh=plsc.VectorSubcoreMesh(core_axis_name="c", subcore_axis_name="s"),
        out_type=jax.ShapeDtypeStruct((NG, GD), jnp.float32),
        scratch_types=[
            pltpu.VMEM((NCH, CH), jnp.int32),
            pltpu.VMEM((RPW, GD), jnp.float32),
            pltpu.SemaphoreType.DMA,
        ],
        compiler_params=pltpu.CompilerParams(use_tc_tiling_on_sc=False),
    )
    def _sc_gather(table_hbm, fidx_hbm, out_hbm, idx_v, rows_v, sem):
        wid = lax.axis_index("s") * _NC + lax.axis_index("c")
        pltpu.sync_copy(fidx_hbm.at[pl.ds(wid * NCH, NCH)], idx_v)
        copies = []
        for j in range(NCH):
            copies.append(
                pltpu.async_copy(table_hbm.at[idx_v.at[j]],
                                 rows_v.at[pl.ds(j * CH, CH)], sem))
        for c in copies:
            c.wait()
        pltpu.sync_copy(rows_v, out_hbm.at[pl.ds(wid * RPW, RPW)])

    return _sc_gather


def kernel(z_e, codebooks):
    zt = z_e.reshape(N, G, GD).transpose(1, 0, 2)
    zn = jnp.sum(zt * zt, axis=2).reshape(G * NB, 1, TN)
    cbn = jnp.sum(codebooks * codebooks, axis=2).reshape(G, 1, K)
    idx_g, fidx_g, loss = _run_argmin(zt, codebooks, zn, cbn)
    idx = idx_g.reshape(G, N).T
    fidx = fidx_g.reshape(G, N).T.reshape(_NW * NCH, CH)
    table = codebooks.reshape(G * K, GD)
    zq = _make_sc_gather()(table, fidx)
    return (zq.reshape(B, T, D), loss[0, 0], idx.reshape(B, T, G))


# speedup vs baseline: 1.7903x; 1.0054x over previous
import functools

import jax
import jax.numpy as jnp
from jax import lax
from jax.experimental import pallas as pl
from jax.experimental.pallas import tpu as pltpu
from jax.experimental.pallas import tpu_sc as plsc

B, T, D = 8, 1024, 256
G, K = 8, 8192
GD = D // G
N = B * T
TN = 1024
NB = N // TN
TK = 1024
NKC = K // TK


def _vq_argmin_kernel(zt_ref, cb_ref, zn_ref, cbn_ref, idx_ref, fidx_ref,
                      loss_ref):
    g = pl.program_id(0)
    nb = pl.program_id(1)
    z = zt_ref[0]
    zb = z.astype(jnp.bfloat16)
    zn = zn_ref[0, 0, :][None, :]

    def body(c, carry):
        rmin, rarg = carry
        cbc = cb_ref[0, pl.ds(c * TK, TK), :]
        prod = lax.dot_general(cbc.astype(jnp.bfloat16), zb,
                               (((1,), (1,)), ((), ())),
                               preferred_element_type=jnp.float32)
        cbn = cbn_ref[0, 0, pl.ds(c * TK, TK)][:, None]
        dist = (zn - 2.0 * prod) + cbn
        m = jnp.min(dist, axis=0)
        a = jnp.argmin(dist, axis=0).astype(jnp.int32) + c * TK
        rmin = jnp.where(c == NKC // 2,
                         rmin.astype(jnp.bfloat16).astype(jnp.float32), rmin)
        better = m < rmin
        return (jnp.where(better, m, rmin), jnp.where(better, a, rarg))

    rmin0 = jnp.full((TN,), jnp.inf, jnp.float32)
    rarg0 = jnp.zeros((TN,), jnp.int32)
    carry = (rmin0, rarg0)
    for c in range(NKC):
        carry = body(c, carry)
    rmin, rarg = carry

    idx_ref[0, 0, :] = rarg
    fidx_ref[0, 0, :] = rarg + g * K

    first = jnp.logical_and(g == 0, nb == 0)
    last = jnp.logical_and(g == G - 1, nb == NB - 1)
    prev = jnp.where(first, jnp.zeros((1, 1), jnp.float32), loss_ref[...])
    total = prev + jnp.sum(rmin)
    loss_ref[...] = jnp.where(last, total * (2.0 / (N * GD)), total)


def _run_argmin(zt, codebooks, zn, cbn):
    return pl.pallas_call(
        _vq_argmin_kernel,
        grid=(G, NB),
        in_specs=[
            pl.BlockSpec((1, TN, GD), lambda g, nb: (g, nb, 0)),
            pl.BlockSpec((1, K, GD), lambda g, nb: (g, 0, 0)),
            pl.BlockSpec((1, 1, TN), lambda g, nb: (g * NB + nb, 0, 0)),
            pl.BlockSpec((1, 1, K), lambda g, nb: (g, 0, 0)),
        ],
        out_specs=[
            pl.BlockSpec((1, 1, TN), lambda g, nb: (g * NB + nb, 0, 0)),
            pl.BlockSpec((1, 1, TN), lambda g, nb: (g * NB + nb, 0, 0)),
            pl.BlockSpec((1, 1), lambda g, nb: (0, 0)),
        ],
        out_shape=[
            jax.ShapeDtypeStruct((G * NB, 1, TN), jnp.int32),
            jax.ShapeDtypeStruct((G * NB, 1, TN), jnp.int32),
            jax.ShapeDtypeStruct((1, 1), jnp.float32),
        ],
    )(zt, codebooks, zn, cbn)


NG = N * G
_NC = 2
_NW = 32
RPW = NG // _NW
CH = 128
NCH = RPW // CH


@functools.cache
def _make_sc_gather():
    @functools.partial(
        pl.kernel,
        mesh=plsc.VectorSubcoreMesh(core_axis_name="c", subcore_axis_name="s"),
        out_type=jax.ShapeDtypeStruct((NG, GD), jnp.float32),
        scratch_types=[
            pltpu.VMEM((NCH, CH), jnp.int32),
            pltpu.VMEM((RPW, GD), jnp.float32),
            pltpu.SemaphoreType.DMA,
        ],
        compiler_params=pltpu.CompilerParams(use_tc_tiling_on_sc=False),
    )
    def _sc_gather(table_hbm, fidx_hbm, out_hbm, idx_v, rows_v, sem):
        wid = lax.axis_index("s") * _NC + lax.axis_index("c")
        pltpu.sync_copy(fidx_hbm.at[pl.ds(wid * NCH, NCH)], idx_v)
        copies = []
        for j in range(NCH):
            copies.append(
                pltpu.async_copy(table_hbm.at[idx_v.at[j]],
                                 rows_v.at[pl.ds(j * CH, CH)], sem))
        for c in copies:
            c.wait()
        pltpu.sync_copy(rows_v, out_hbm.at[pl.ds(wid * RPW, RPW)])

    return _sc_gather


def kernel(z_e, codebooks):
    zt = z_e.reshape(N, G, GD).transpose(1, 0, 2)
    zn = jnp.sum(zt * zt, axis=2).reshape(G * NB, 1, TN)
    cbn = jnp.sum(codebooks * codebooks, axis=2).reshape(G, 1, K)
    idx_g, fidx_g, loss = _run_argmin(zt, codebooks, zn, cbn)
    idx = idx_g.reshape(G, N).T
    fidx = fidx_g.reshape(G, N).T.reshape(_NW * NCH, CH)
    table = codebooks.reshape(G * K, GD)
    zq = _make_sc_gather()(table, fidx)
    return (zq.reshape(B, T, D), loss[0, 0], idx.reshape(B, T, G))


# speedup vs baseline: 1.9869x; 1.1098x over previous
import functools

import jax
import jax.numpy as jnp
from jax import lax
from jax.experimental import pallas as pl
from jax.experimental.pallas import tpu as pltpu
from jax.experimental.pallas import tpu_sc as plsc

B, T, D = 8, 1024, 256
G, K = 8, 8192
GD = D // G
N = B * T
TN = 1024
NB = N // TN
TK = 1024
NKC = K // TK


def _vq_argmin_kernel(zt_ref, cb_ref, zn_ref, cbn_ref, idx_ref, fidx_ref,
                      loss_ref):
    g = pl.program_id(0)
    nb = pl.program_id(1)
    z = zt_ref[0]
    zb = z.astype(jnp.bfloat16)
    zn = zn_ref[0, 0, :][None, :]

    def body(c, carry):
        rmin, rarg = carry
        cbc = cb_ref[0, pl.ds(c * TK, TK), :]
        prod2 = lax.dot_general(cbc.astype(jnp.bfloat16) * jnp.bfloat16(-2.0),
                                zb, (((1,), (1,)), ((), ())),
                                preferred_element_type=jnp.float32)
        cbn = cbn_ref[0, 0, pl.ds(c * TK, TK)][:, None]
        dist = (zn + prod2) + cbn
        m = jnp.min(dist, axis=0)
        a = jnp.argmin(dist, axis=0).astype(jnp.int32) + c * TK
        rmin = jnp.where(c == NKC // 2,
                         rmin.astype(jnp.bfloat16).astype(jnp.float32), rmin)
        better = m < rmin
        return (jnp.where(better, m, rmin), jnp.where(better, a, rarg))

    rmin0 = jnp.full((TN,), jnp.inf, jnp.float32)
    rarg0 = jnp.zeros((TN,), jnp.int32)
    carry = (rmin0, rarg0)
    for c in range(NKC):
        carry = body(c, carry)
    rmin, rarg = carry

    idx_ref[0, 0, :] = rarg
    fidx_ref[0, 0, :] = rarg + g * K

    first = jnp.logical_and(g == 0, nb == 0)
    last = jnp.logical_and(g == G - 1, nb == NB - 1)
    prev = jnp.where(first, jnp.zeros((1, 1), jnp.float32), loss_ref[...])
    total = prev + jnp.sum(rmin)
    loss_ref[...] = jnp.where(last, total * (2.0 / (N * GD)), total)


def _run_argmin(zt, codebooks, zn, cbn):
    return pl.pallas_call(
        _vq_argmin_kernel,
        grid=(G, NB),
        in_specs=[
            pl.BlockSpec((1, TN, GD), lambda g, nb: (g, nb, 0)),
            pl.BlockSpec((1, K, GD), lambda g, nb: (g, 0, 0)),
            pl.BlockSpec((1, 1, TN), lambda g, nb: (g * NB + nb, 0, 0)),
            pl.BlockSpec((1, 1, K), lambda g, nb: (g, 0, 0)),
        ],
        out_specs=[
            pl.BlockSpec((1, 1, TN), lambda g, nb: (g * NB + nb, 0, 0)),
            pl.BlockSpec((1, 1, TN), lambda g, nb: (g * NB + nb, 0, 0)),
            pl.BlockSpec((1, 1), lambda g, nb: (0, 0)),
        ],
        out_shape=[
            jax.ShapeDtypeStruct((G * NB, 1, TN), jnp.int32),
            jax.ShapeDtypeStruct((G * NB, 1, TN), jnp.int32),
            jax.ShapeDtypeStruct((1, 1), jnp.float32),
        ],
    )(zt, codebooks, zn, cbn)


NG = N * G
_NC = 2
_NW = 32
RPW = NG // _NW
CH = 128
NCH = RPW // CH


@functools.cache
def _make_sc_gather():
    @functools.partial(
        pl.kernel,
        mesh=plsc.VectorSubcoreMesh(core_axis_name="c", subcore_axis_name="s"),
        out_type=jax.ShapeDtypeStruct((NG, GD), jnp.float32),
        scratch_types=[
            pltpu.VMEM((NCH, CH), jnp.int32),
            pltpu.VMEM((RPW, GD), jnp.float32),
            pltpu.SemaphoreType.DMA,
        ],
        compiler_params=pltpu.CompilerParams(use_tc_tiling_on_sc=False),
    )
    def _sc_gather(table_hbm, fidx_hbm, out_hbm, idx_v, rows_v, sem):
        wid = lax.axis_index("s") * _NC + lax.axis_index("c")
        pltpu.sync_copy(fidx_hbm.at[pl.ds(wid * NCH, NCH)], idx_v)
        copies = []
        for j in range(NCH):
            copies.append(
                pltpu.async_copy(table_hbm.at[idx_v.at[j]],
                                 rows_v.at[pl.ds(j * CH, CH)], sem))
        for c in copies:
            c.wait()
        pltpu.sync_copy(rows_v, out_hbm.at[pl.ds(wid * RPW, RPW)])

    return _sc_gather


def kernel(z_e, codebooks):
    zt = z_e.reshape(N, G, GD).transpose(1, 0, 2)
    zn = jnp.sum(zt * zt, axis=2).reshape(G * NB, 1, TN)
    cbn = jnp.sum(codebooks * codebooks, axis=2).reshape(G, 1, K)
    idx_g, fidx_g, loss = _run_argmin(zt, codebooks, zn, cbn)
    idx = idx_g.reshape(G, N).T
    fidx = fidx_g.reshape(G, N).T.reshape(_NW * NCH, CH)
    table = codebooks.reshape(G * K, GD)
    zq = _make_sc_gather()(table, fidx)
    return (zq.reshape(B, T, D), loss[0, 0], idx.reshape(B, T, G))
